# Initial kernel scaffold; baseline (speedup 1.0000x reference)
import jax
import jax.numpy as jnp
from jax.experimental import pallas as pl


def kernel(node_features, edge_index, edge_type, node_mask, handcrafted_features, ne_W, ne_b, ne_g, ne_beta, edge_emb, edge_scale, gine_eps, gine_W1, gine_b1, gine_g1, gine_beta1, gine_W2, gine_b2, gine_g2, gine_beta2, ln_g, ln_b, vn_W1, vn_b1, vn_g1, vn_beta1, vn_W2, vn_b2, vn_g2, vn_beta2, vn_gate, vn_init, gp_W, gp_b, gp_g, gp_beta, fe_W1, fe_b1, fe_g1, fe_beta1, fe_W2, fe_b2, fe_g2, fe_beta2, cl_W1, cl_b1, cl_g, cl_beta, cl_W2, cl_b2):
    raise NotImplementedError("write your pallas kernel here")


# speedup vs baseline: 16.3912x; 16.3912x over previous
import math

import jax
import jax.numpy as jnp
from jax.experimental import pallas as pl
from jax.experimental.pallas import tpu as pltpu

_BN_C = 1.0 / math.sqrt(1.0 + 1e-5)


def _gnn_body(ei_ref, et_ref, mask_ref, nf_ref,
              ne_W_ref, ne_b_ref, ne_g_ref, ne_beta_ref,
              We_ref, vn_init_ref,
              eps_ref, gate_ref,
              gW1_ref, gb1_ref, gg1_ref, gbeta1_ref,
              gW2_ref, gb2_ref, gg2_ref, gbeta2_ref,
              lng_ref, lnb_ref,
              vW1_ref, vb1_ref, vg1_ref, vbeta1_ref,
              vW2_ref, vb2_ref, vg2_ref, vbeta2_ref,
              gr_ref):
    N = nf_ref.shape[1]
    E = ei_ref.shape[2]
    H = ne_W_ref.shape[1]
    L = gW1_ref.shape[0]
    NET = We_ref.shape[0]
    f32 = jnp.float32

    nf = nf_ref[0]
    maskcol = mask_ref[0]

    z = jnp.dot(nf, ne_W_ref[...], preferred_element_type=f32) + ne_b_ref[...]
    h = jnp.maximum(ne_g_ref[...] * (z * _BN_C) + ne_beta_ref[...], 0.0)

    src = ei_ref[0, 0, :].reshape(1, E)
    dst = ei_ref[0, 1, :].reshape(1, E)
    iota_ne = jax.lax.broadcasted_iota(jnp.int32, (N, E), 0)
    St = (iota_ne == src).astype(f32)
    Dt = (iota_ne == dst).astype(f32)

    et = et_ref[0, 0, :].reshape(1, E)
    iota_te = jax.lax.broadcasted_iota(jnp.int32, (NET, E), 0)
    ET = (iota_te == et).astype(f32)
    ea = jax.lax.dot_general(ET, We_ref[...], (((0,), (0,)), ((), ())),
                             preferred_element_type=f32)

    vn = vn_init_ref[...]
    segs = [jnp.sum(h * maskcol, axis=0)]

    for i in range(L):
        hsrc = jax.lax.dot_general(St, h, (((0,), (0,)), ((), ())),
                                   preferred_element_type=f32)
        msg = jnp.maximum(hsrc + ea, 0.0)
        agg = jnp.dot(Dt, msg, preferred_element_type=f32)

        hn = eps_ref[i] * h + agg
        z = jnp.dot(hn, gW1_ref[i], preferred_element_type=f32) + gb1_ref[i]
        z = jnp.maximum(gg1_ref[i] * (z * _BN_C) + gbeta1_ref[i], 0.0)
        z = jnp.dot(z, gW2_ref[i], preferred_element_type=f32) + gb2_ref[i]
        z = gg2_ref[i] * (z * _BN_C) + gbeta2_ref[i]
        z = z * maskcol

        r = h + z
        m = jnp.mean(r, axis=1, keepdims=True)
        d = r - m
        v = jnp.mean(d * d, axis=1, keepdims=True)
        hln = d * jax.lax.rsqrt(v + 1e-5) * lng_ref[i] + lnb_ref[i]

        ns = jnp.sum(hln * maskcol, axis=0).reshape(1, H)
        vn_new = vn + ns
        y = jnp.dot(vn_new, vW1_ref[i], preferred_element_type=f32) + vb1_ref[i]
        y = jnp.maximum(vg1_ref[i] * (y * _BN_C) + vbeta1_ref[i], 0.0)
        y = jnp.dot(y, vW2_ref[i], preferred_element_type=f32) + vb2_ref[i]
        y = vg2_ref[i] * (y * _BN_C) + vbeta2_ref[i]
        vn_new = y + vn

        h = (hln + gate_ref[i] * vn_new) * maskcol
        vn = vn_new
        segs.append(jnp.sum(h * maskcol, axis=0))

    gr_ref[0, 0, :] = jnp.concatenate(segs, axis=0)


def _head_body(gr_ref, hc_ref,
               gpW_ref, gpb_ref, gpg_ref, gpbe_ref,
               feW1_ref, feb1_ref, feg1_ref, febe1_ref,
               feW2_ref, feb2_ref, feg2_ref, febe2_ref,
               clW1_ref, clb1_ref, clg_ref, clbe_ref,
               clW2_ref, clb2_ref, out_ref):
    f32 = jnp.float32
    x = jnp.dot(gr_ref[...], gpW_ref[...], preferred_element_type=f32) + gpb_ref[...]
    x = jnp.maximum(gpg_ref[...] * (x * _BN_C) + gpbe_ref[...], 0.0)

    f = jnp.dot(hc_ref[...], feW1_ref[...], preferred_element_type=f32) + feb1_ref[...]
    f = jnp.maximum(feg1_ref[...] * (f * _BN_C) + febe1_ref[...], 0.0)
    f = jnp.dot(f, feW2_ref[...], preferred_element_type=f32) + feb2_ref[...]
    f = jnp.maximum(feg2_ref[...] * (f * _BN_C) + febe2_ref[...], 0.0)

    comb = jnp.concatenate([x, f], axis=1)
    y = jnp.dot(comb, clW1_ref[...], preferred_element_type=f32) + clb1_ref[...]
    y = jnp.maximum(clg_ref[...] * (y * _BN_C) + clbe_ref[...], 0.0)
    out_ref[...] = jnp.dot(y, clW2_ref[...], preferred_element_type=f32) + clb2_ref[...]


def kernel(node_features, edge_index, edge_type, node_mask, handcrafted_features,
           ne_W, ne_b, ne_g, ne_beta, edge_emb, edge_scale, gine_eps,
           gine_W1, gine_b1, gine_g1, gine_beta1,
           gine_W2, gine_b2, gine_g2, gine_beta2,
           ln_g, ln_b,
           vn_W1, vn_b1, vn_g1, vn_beta1,
           vn_W2, vn_b2, vn_g2, vn_beta2,
           vn_gate, vn_init,
           gp_W, gp_b, gp_g, gp_beta,
           fe_W1, fe_b1, fe_g1, fe_beta1,
           fe_W2, fe_b2, fe_g2, fe_beta2,
           cl_W1, cl_b1, cl_g, cl_beta, cl_W2, cl_b2):
    B, N, FEAT = node_features.shape
    E = edge_index.shape[2]
    H = ne_W.shape[1]
    L = gine_eps.shape[0]
    f32 = jnp.float32

    mask3 = node_mask.astype(f32).reshape(B, N, 1)
    et3 = edge_type.astype(jnp.int32).reshape(B, 1, E)
    ei = edge_index.astype(jnp.int32)
    We = edge_emb * edge_scale[:, None]
    eps1p = (1.0 + gine_eps).astype(f32)
    gate = jax.nn.sigmoid(vn_gate).astype(f32)

    def full(a):
        nd = a.ndim
        return pl.BlockSpec(a.shape, lambda b, _n=nd: (0,) * _n)

    smem = pl.BlockSpec(memory_space=pltpu.SMEM)

    gr = pl.pallas_call(
        _gnn_body,
        grid=(B,),
        in_specs=[
            pl.BlockSpec((1, 2, E), lambda b: (b, 0, 0)),
            pl.BlockSpec((1, 1, E), lambda b: (b, 0, 0)),
            pl.BlockSpec((1, N, 1), lambda b: (b, 0, 0)),
            pl.BlockSpec((1, N, FEAT), lambda b: (b, 0, 0)),
            full(ne_W), full(ne_b), full(ne_g), full(ne_beta),
            full(We), full(vn_init),
            smem, smem,
            full(gine_W1), full(gine_b1), full(gine_g1), full(gine_beta1),
            full(gine_W2), full(gine_b2), full(gine_g2), full(gine_beta2),
            full(ln_g), full(ln_b),
            full(vn_W1), full(vn_b1), full(vn_g1), full(vn_beta1),
            full(vn_W2), full(vn_b2), full(vn_g2), full(vn_beta2),
        ],
        out_specs=pl.BlockSpec((1, 1, (L + 1) * H), lambda b: (b, 0, 0)),
        out_shape=jax.ShapeDtypeStruct((B, 1, (L + 1) * H), f32),
    )(ei, et3, mask3, node_features,
      ne_W, ne_b, ne_g, ne_beta, We, vn_init, eps1p, gate,
      gine_W1, gine_b1, gine_g1, gine_beta1,
      gine_W2, gine_b2, gine_g2, gine_beta2,
      ln_g, ln_b,
      vn_W1, vn_b1, vn_g1, vn_beta1,
      vn_W2, vn_b2, vn_g2, vn_beta2)

    gr2 = gr.reshape(B, (L + 1) * H)

    logits = pl.pallas_call(
        _head_body,
        out_shape=jax.ShapeDtypeStruct((B, cl_W2.shape[1]), f32),
    )(gr2, handcrafted_features,
      gp_W, gp_b, gp_g, gp_beta,
      fe_W1, fe_b1, fe_g1, fe_beta1,
      fe_W2, fe_b2, fe_g2, fe_beta2,
      cl_W1, cl_b1, cl_g, cl_beta, cl_W2, cl_b2)

    return logits


# speedup vs baseline: 16.7170x; 1.0199x over previous
import math

import jax
import jax.numpy as jnp
from jax.experimental import pallas as pl
from jax.experimental.pallas import tpu as pltpu

_BN_C = 1.0 / math.sqrt(1.0 + 1e-5)
_G = 2


def _gnn_body(ei_ref, et_ref, mask_ref, nf_ref,
              ne_W_ref, ne_b_ref, ne_g_ref, ne_beta_ref,
              We_ref, vn_init_ref,
              eps_ref, gate_ref,
              gW1_ref, gb1_ref, gg1_ref, gbeta1_ref,
              gW2_ref, gb2_ref, gg2_ref, gbeta2_ref,
              lng_ref, lnb_ref,
              vW1_ref, vb1_ref, vg1_ref, vbeta1_ref,
              vW2_ref, vb2_ref, vg2_ref, vbeta2_ref,
              gr_ref):
    N = nf_ref.shape[1]
    E = ei_ref.shape[2]
    H = ne_W_ref.shape[1]
    L = gW1_ref.shape[0]
    NET = We_ref.shape[0]
    f32 = jnp.float32

    for g in range(_G):
        nf = nf_ref[g]
        maskcol = mask_ref[g]

        z = jnp.dot(nf, ne_W_ref[...], preferred_element_type=f32) + ne_b_ref[...]
        h = jnp.maximum(ne_g_ref[...] * (z * _BN_C) + ne_beta_ref[...], 0.0)

        src = ei_ref[g, 0, :].reshape(1, E)
        dst = ei_ref[g, 1, :].reshape(1, E)
        iota_ne = jax.lax.broadcasted_iota(jnp.int32, (N, E), 0)
        St = (iota_ne == src).astype(f32)
        Dt = (iota_ne == dst).astype(f32)

        et = et_ref[g, 0, :].reshape(1, E)
        iota_te = jax.lax.broadcasted_iota(jnp.int32, (NET, E), 0)
        ET = (iota_te == et).astype(f32)
        ea = jax.lax.dot_general(ET, We_ref[...], (((0,), (0,)), ((), ())),
                                 preferred_element_type=f32)

        vn = vn_init_ref[...]
        segs = [jnp.sum(h * maskcol, axis=0)]

        for i in range(L):
            hsrc = jax.lax.dot_general(St, h, (((0,), (0,)), ((), ())),
                                       preferred_element_type=f32)
            msg = jnp.maximum(hsrc + ea, 0.0)
            agg = jnp.dot(Dt, msg, preferred_element_type=f32)

            hn = eps_ref[i] * h + agg
            z = jnp.dot(hn, gW1_ref[i], preferred_element_type=f32) + gb1_ref[i]
            z = jnp.maximum(gg1_ref[i] * (z * _BN_C) + gbeta1_ref[i], 0.0)
            z = jnp.dot(z, gW2_ref[i], preferred_element_type=f32) + gb2_ref[i]
            z = gg2_ref[i] * (z * _BN_C) + gbeta2_ref[i]
            z = z * maskcol

            r = h + z
            m = jnp.mean(r, axis=1, keepdims=True)
            d = r - m
            v = jnp.mean(d * d, axis=1, keepdims=True)
            hln = d * jax.lax.rsqrt(v + 1e-5) * lng_ref[i] + lnb_ref[i]

            ns = jnp.sum(hln * maskcol, axis=0).reshape(1, H)
            vn_new = vn + ns
            y = jnp.dot(vn_new, vW1_ref[i], preferred_element_type=f32) + vb1_ref[i]
            y = jnp.maximum(vg1_ref[i] * (y * _BN_C) + vbeta1_ref[i], 0.0)
            y = jnp.dot(y, vW2_ref[i], preferred_element_type=f32) + vb2_ref[i]
            y = vg2_ref[i] * (y * _BN_C) + vbeta2_ref[i]
            vn_new = y + vn

            h = (hln + gate_ref[i] * vn_new) * maskcol
            vn = vn_new
            segs.append(jnp.sum(h * maskcol, axis=0))

        gr_ref[g, 0, :] = jnp.concatenate(segs, axis=0)


def _head_body(gr_ref, hc_ref,
               gpW_ref, gpb_ref, gpg_ref, gpbe_ref,
               feW1_ref, feb1_ref, feg1_ref, febe1_ref,
               feW2_ref, feb2_ref, feg2_ref, febe2_ref,
               clW1_ref, clb1_ref, clg_ref, clbe_ref,
               clW2_ref, clb2_ref, out_ref):
    f32 = jnp.float32
    x = jnp.dot(gr_ref[...], gpW_ref[...], preferred_element_type=f32) + gpb_ref[...]
    x = jnp.maximum(gpg_ref[...] * (x * _BN_C) + gpbe_ref[...], 0.0)

    f = jnp.dot(hc_ref[...], feW1_ref[...], preferred_element_type=f32) + feb1_ref[...]
    f = jnp.maximum(feg1_ref[...] * (f * _BN_C) + febe1_ref[...], 0.0)
    f = jnp.dot(f, feW2_ref[...], preferred_element_type=f32) + feb2_ref[...]
    f = jnp.maximum(feg2_ref[...] * (f * _BN_C) + febe2_ref[...], 0.0)

    comb = jnp.concatenate([x, f], axis=1)
    y = jnp.dot(comb, clW1_ref[...], preferred_element_type=f32) + clb1_ref[...]
    y = jnp.maximum(clg_ref[...] * (y * _BN_C) + clbe_ref[...], 0.0)
    out_ref[...] = jnp.dot(y, clW2_ref[...], preferred_element_type=f32) + clb2_ref[...]


def kernel(node_features, edge_index, edge_type, node_mask, handcrafted_features,
           ne_W, ne_b, ne_g, ne_beta, edge_emb, edge_scale, gine_eps,
           gine_W1, gine_b1, gine_g1, gine_beta1,
           gine_W2, gine_b2, gine_g2, gine_beta2,
           ln_g, ln_b,
           vn_W1, vn_b1, vn_g1, vn_beta1,
           vn_W2, vn_b2, vn_g2, vn_beta2,
           vn_gate, vn_init,
           gp_W, gp_b, gp_g, gp_beta,
           fe_W1, fe_b1, fe_g1, fe_beta1,
           fe_W2, fe_b2, fe_g2, fe_beta2,
           cl_W1, cl_b1, cl_g, cl_beta, cl_W2, cl_b2):
    B, N, FEAT = node_features.shape
    E = edge_index.shape[2]
    H = ne_W.shape[1]
    L = gine_eps.shape[0]
    f32 = jnp.float32

    mask3 = node_mask.astype(f32).reshape(B, N, 1)
    et3 = edge_type.astype(jnp.int32).reshape(B, 1, E)
    ei = edge_index.astype(jnp.int32)
    We = edge_emb * edge_scale[:, None]
    eps1p = (1.0 + gine_eps).astype(f32)
    gate = jax.nn.sigmoid(vn_gate).astype(f32)

    def full(a):
        nd = a.ndim
        return pl.BlockSpec(a.shape, lambda b, _n=nd: (0,) * _n)

    smem = pl.BlockSpec(memory_space=pltpu.SMEM)

    gr = pl.pallas_call(
        _gnn_body,
        grid=(B // _G,),
        in_specs=[
            pl.BlockSpec((_G, 2, E), lambda b: (b, 0, 0)),
            pl.BlockSpec((_G, 1, E), lambda b: (b, 0, 0)),
            pl.BlockSpec((_G, N, 1), lambda b: (b, 0, 0)),
            pl.BlockSpec((_G, N, FEAT), lambda b: (b, 0, 0)),
            full(ne_W), full(ne_b), full(ne_g), full(ne_beta),
            full(We), full(vn_init),
            smem, smem,
            full(gine_W1), full(gine_b1), full(gine_g1), full(gine_beta1),
            full(gine_W2), full(gine_b2), full(gine_g2), full(gine_beta2),
            full(ln_g), full(ln_b),
            full(vn_W1), full(vn_b1), full(vn_g1), full(vn_beta1),
            full(vn_W2), full(vn_b2), full(vn_g2), full(vn_beta2),
        ],
        out_specs=pl.BlockSpec((_G, 1, (L + 1) * H), lambda b: (b, 0, 0)),
        out_shape=jax.ShapeDtypeStruct((B, 1, (L + 1) * H), f32),
    )(ei, et3, mask3, node_features,
      ne_W, ne_b, ne_g, ne_beta, We, vn_init, eps1p, gate,
      gine_W1, gine_b1, gine_g1, gine_beta1,
      gine_W2, gine_b2, gine_g2, gine_beta2,
      ln_g, ln_b,
      vn_W1, vn_b1, vn_g1, vn_beta1,
      vn_W2, vn_b2, vn_g2, vn_beta2)

    gr2 = gr.reshape(B, (L + 1) * H)

    logits = pl.pallas_call(
        _head_body,
        out_shape=jax.ShapeDtypeStruct((B, cl_W2.shape[1]), f32),
    )(gr2, handcrafted_features,
      gp_W, gp_b, gp_g, gp_beta,
      fe_W1, fe_b1, fe_g1, fe_beta1,
      fe_W2, fe_b2, fe_g2, fe_beta2,
      cl_W1, cl_b1, cl_g, cl_beta, cl_W2, cl_b2)

    return logits


# speedup vs baseline: 16.8573x; 1.0084x over previous
import math

import jax
import jax.numpy as jnp
from jax.experimental import pallas as pl
from jax.experimental.pallas import tpu as pltpu

_BN_C = 1.0 / math.sqrt(1.0 + 1e-5)
_G = 2


def _gnn_body(ei_ref, et_ref, mask_ref, nf_ref,
              ne_W_ref, ne_b_ref, ne_g_ref, ne_beta_ref,
              We_ref, vn_init_ref,
              eps_ref, gate_ref,
              gW1_ref, gb1_ref, gg1_ref, gbeta1_ref,
              gW2_ref, gb2_ref, gg2_ref, gbeta2_ref,
              lng_ref, lnb_ref,
              vW1_ref, vb1_ref, vg1_ref, vbeta1_ref,
              vW2_ref, vb2_ref, vg2_ref, vbeta2_ref,
              gr_ref):
    N = nf_ref.shape[1]
    E = ei_ref.shape[2]
    H = ne_W_ref.shape[1]
    L = gW1_ref.shape[0]
    NET = We_ref.shape[0]
    f32 = jnp.float32

    for g in range(_G):
        nf = nf_ref[g]
        maskcol = mask_ref[g]

        z = jnp.dot(nf, ne_W_ref[...], preferred_element_type=f32) + ne_b_ref[...]
        h = jnp.maximum(ne_g_ref[...] * (z * _BN_C) + ne_beta_ref[...], 0.0)

        src = ei_ref[g, 0, :].reshape(1, E)
        dst = ei_ref[g, 1, :].reshape(1, E)
        iota_ne = jax.lax.broadcasted_iota(jnp.int32, (N, E), 0)
        St = (iota_ne == src).astype(jnp.bfloat16)
        Dt = (iota_ne == dst).astype(jnp.bfloat16)

        et = et_ref[g, 0, :].reshape(1, E)
        iota_te = jax.lax.broadcasted_iota(jnp.int32, (NET, E), 0)
        ET = (iota_te == et).astype(f32)
        ea = jax.lax.dot_general(ET, We_ref[...], (((0,), (0,)), ((), ())),
                                 preferred_element_type=f32)

        vn = vn_init_ref[...]
        segs = [jnp.sum(h * maskcol, axis=0)]

        for i in range(L):
            hsrc = jax.lax.dot_general(St, h.astype(jnp.bfloat16),
                                       (((0,), (0,)), ((), ())),
                                       preferred_element_type=f32)
            msg = jnp.maximum(hsrc + ea, 0.0)
            agg = jnp.dot(Dt, msg.astype(jnp.bfloat16),
                          preferred_element_type=f32)

            hn = eps_ref[i] * h + agg
            z = jnp.dot(hn, gW1_ref[i], preferred_element_type=f32) + gb1_ref[i]
            z = jnp.maximum(gg1_ref[i] * (z * _BN_C) + gbeta1_ref[i], 0.0)
            z = jnp.dot(z, gW2_ref[i], preferred_element_type=f32) + gb2_ref[i]
            z = gg2_ref[i] * (z * _BN_C) + gbeta2_ref[i]
            z = z * maskcol

            r = h + z
            m = jnp.mean(r, axis=1, keepdims=True)
            d = r - m
            v = jnp.mean(d * d, axis=1, keepdims=True)
            hln = d * jax.lax.rsqrt(v + 1e-5) * lng_ref[i] + lnb_ref[i]

            ns = jnp.sum(hln * maskcol, axis=0).reshape(1, H)
            vn_new = vn + ns
            y = jnp.dot(vn_new, vW1_ref[i], preferred_element_type=f32) + vb1_ref[i]
            y = jnp.maximum(vg1_ref[i] * (y * _BN_C) + vbeta1_ref[i], 0.0)
            y = jnp.dot(y, vW2_ref[i], preferred_element_type=f32) + vb2_ref[i]
            y = vg2_ref[i] * (y * _BN_C) + vbeta2_ref[i]
            vn_new = y + vn

            h = (hln + gate_ref[i] * vn_new) * maskcol
            vn = vn_new
            segs.append(jnp.sum(h * maskcol, axis=0))

        gr_ref[g, 0, :] = jnp.concatenate(segs, axis=0)


def _head_body(gr_ref, hc_ref,
               gpW_ref, gpb_ref, gpg_ref, gpbe_ref,
               feW1_ref, feb1_ref, feg1_ref, febe1_ref,
               feW2_ref, feb2_ref, feg2_ref, febe2_ref,
               clW1_ref, clb1_ref, clg_ref, clbe_ref,
               clW2_ref, clb2_ref, out_ref):
    f32 = jnp.float32
    x = jnp.dot(gr_ref[...], gpW_ref[...], preferred_element_type=f32) + gpb_ref[...]
    x = jnp.maximum(gpg_ref[...] * (x * _BN_C) + gpbe_ref[...], 0.0)

    f = jnp.dot(hc_ref[...], feW1_ref[...], preferred_element_type=f32) + feb1_ref[...]
    f = jnp.maximum(feg1_ref[...] * (f * _BN_C) + febe1_ref[...], 0.0)
    f = jnp.dot(f, feW2_ref[...], preferred_element_type=f32) + feb2_ref[...]
    f = jnp.maximum(feg2_ref[...] * (f * _BN_C) + febe2_ref[...], 0.0)

    comb = jnp.concatenate([x, f], axis=1)
    y = jnp.dot(comb, clW1_ref[...], preferred_element_type=f32) + clb1_ref[...]
    y = jnp.maximum(clg_ref[...] * (y * _BN_C) + clbe_ref[...], 0.0)
    out_ref[...] = jnp.dot(y, clW2_ref[...], preferred_element_type=f32) + clb2_ref[...]


def kernel(node_features, edge_index, edge_type, node_mask, handcrafted_features,
           ne_W, ne_b, ne_g, ne_beta, edge_emb, edge_scale, gine_eps,
           gine_W1, gine_b1, gine_g1, gine_beta1,
           gine_W2, gine_b2, gine_g2, gine_beta2,
           ln_g, ln_b,
           vn_W1, vn_b1, vn_g1, vn_beta1,
           vn_W2, vn_b2, vn_g2, vn_beta2,
           vn_gate, vn_init,
           gp_W, gp_b, gp_g, gp_beta,
           fe_W1, fe_b1, fe_g1, fe_beta1,
           fe_W2, fe_b2, fe_g2, fe_beta2,
           cl_W1, cl_b1, cl_g, cl_beta, cl_W2, cl_b2):
    B, N, FEAT = node_features.shape
    E = edge_index.shape[2]
    H = ne_W.shape[1]
    L = gine_eps.shape[0]
    f32 = jnp.float32

    mask3 = node_mask.astype(f32).reshape(B, N, 1)
    et3 = edge_type.astype(jnp.int32).reshape(B, 1, E)
    ei = edge_index.astype(jnp.int32)
    We = edge_emb * edge_scale[:, None]
    eps1p = (1.0 + gine_eps).astype(f32)
    gate = jax.nn.sigmoid(vn_gate).astype(f32)

    def full(a):
        nd = a.ndim
        return pl.BlockSpec(a.shape, lambda b, _n=nd: (0,) * _n)

    smem = pl.BlockSpec(memory_space=pltpu.SMEM)

    gr = pl.pallas_call(
        _gnn_body,
        grid=(B // _G,),
        in_specs=[
            pl.BlockSpec((_G, 2, E), lambda b: (b, 0, 0)),
            pl.BlockSpec((_G, 1, E), lambda b: (b, 0, 0)),
            pl.BlockSpec((_G, N, 1), lambda b: (b, 0, 0)),
            pl.BlockSpec((_G, N, FEAT), lambda b: (b, 0, 0)),
            full(ne_W), full(ne_b), full(ne_g), full(ne_beta),
            full(We), full(vn_init),
            smem, smem,
            full(gine_W1), full(gine_b1), full(gine_g1), full(gine_beta1),
            full(gine_W2), full(gine_b2), full(gine_g2), full(gine_beta2),
            full(ln_g), full(ln_b),
            full(vn_W1), full(vn_b1), full(vn_g1), full(vn_beta1),
            full(vn_W2), full(vn_b2), full(vn_g2), full(vn_beta2),
        ],
        out_specs=pl.BlockSpec((_G, 1, (L + 1) * H), lambda b: (b, 0, 0)),
        out_shape=jax.ShapeDtypeStruct((B, 1, (L + 1) * H), f32),
    )(ei, et3, mask3, node_features,
      ne_W, ne_b, ne_g, ne_beta, We, vn_init, eps1p, gate,
      gine_W1, gine_b1, gine_g1, gine_beta1,
      gine_W2, gine_b2, gine_g2, gine_beta2,
      ln_g, ln_b,
      vn_W1, vn_b1, vn_g1, vn_beta1,
      vn_W2, vn_b2, vn_g2, vn_beta2)

    gr2 = gr.reshape(B, (L + 1) * H)

    logits = pl.pallas_call(
        _head_body,
        out_shape=jax.ShapeDtypeStruct((B, cl_W2.shape[1]), f32),
    )(gr2, handcrafted_features,
      gp_W, gp_b, gp_g, gp_beta,
      fe_W1, fe_b1, fe_g1, fe_beta1,
      fe_W2, fe_b2, fe_g2, fe_beta2,
      cl_W1, cl_b1, cl_g, cl_beta, cl_W2, cl_b2)

    return logits
